# Initial kernel scaffold; baseline (speedup 1.0000x reference)
import jax
import jax.numpy as jnp
from jax.experimental import pallas as pl


def kernel(features, edge_index, edge_values, W_gc0, W_gc1, W_gc2, Wt0, bt0, Wt1, bt1, Wt2, bt2):
    raise NotImplementedError("write your pallas kernel here")


# speedup vs baseline: 1.9467x; 1.9467x over previous
import functools

import jax
import jax.numpy as jnp
from jax import lax
from jax.experimental import pallas as pl
from jax.experimental.pallas import tpu as pltpu
from jax.experimental.pallas import tpu_sc as plsc

N = 10000
NP = 10240
E = 160000
D2 = 128

RB = 1000
NB = N // RB
TILES = 16
EPT = E // TILES
C = 80
NCHUNK = EPT // C
HALF = NP // 2
RPT = HALF // TILES
ZR = 64


def _mm_halves(x, w):
    F = x.shape[1]

    def body(x_ref, w_ref, o_ref):
        o_ref[...] = jnp.dot(x_ref[...], w_ref[...],
                             preferred_element_type=jnp.float32)

    return pl.pallas_call(
        body,
        grid=(2, NB),
        in_specs=[
            pl.BlockSpec((RB, F), lambda c, r: (r, 0)),
            pl.BlockSpec((F, D2), lambda c, r: (0, c)),
        ],
        out_specs=pl.BlockSpec((RB, D2), lambda c, r: (c * NB + r, 0)),
        out_shape=jax.ShapeDtypeStruct((2 * N, D2), jnp.float32),
    )(x, w)


def _trans_mm_halves(h, x, At, b, wg3):
    Hout = At.shape[1]
    F = x.shape[1]

    def body(ha_ref, hb_ref, x_ref, At_ref, b_ref, wg_ref, o_ref):
        At_v = At_ref[...]
        hid = (jnp.dot(ha_ref[0], At_v[:D2], preferred_element_type=jnp.float32)
               + jnp.dot(hb_ref[0], At_v[D2:2 * D2],
                         preferred_element_type=jnp.float32)
               + jnp.dot(x_ref[...], At_v[2 * D2:],
                         preferred_element_type=jnp.float32)
               + b_ref[...])
        hid = jnp.maximum(hid, 0.0)
        o_ref[...] = jnp.dot(hid, wg_ref[0],
                             preferred_element_type=jnp.float32)

    return pl.pallas_call(
        body,
        grid=(2, NB),
        in_specs=[
            pl.BlockSpec((1, RB, D2), lambda c, r: (0, r, 0)),
            pl.BlockSpec((1, RB, D2), lambda c, r: (1, r, 0)),
            pl.BlockSpec((RB, F), lambda c, r: (r, 0)),
            pl.BlockSpec((2 * D2 + F, Hout), lambda c, r: (0, 0)),
            pl.BlockSpec((1, Hout), lambda c, r: (0, 0)),
            pl.BlockSpec((1, Hout, D2), lambda c, r: (c, 0, 0)),
        ],
        out_specs=pl.BlockSpec((RB, D2), lambda c, r: (c * NB + r, 0)),
        out_shape=jax.ShapeDtypeStruct((2 * N, D2), jnp.float32),
    )(h, h, x, At, b, wg3)


def _trans_final(h, x, At, b):
    Hout = At.shape[1]
    F = x.shape[1]

    def body(ha_ref, x_ref, At_ref, b_ref, o_ref):
        At_v = At_ref[...]
        hid = (jnp.dot(ha_ref[0], At_v[:D2], preferred_element_type=jnp.float32)
               + jnp.dot(x_ref[...], At_v[D2:],
                         preferred_element_type=jnp.float32)
               + b_ref[...])
        o_ref[...] = jnp.maximum(hid, 0.0)

    return pl.pallas_call(
        body,
        grid=(NB,),
        in_specs=[
            pl.BlockSpec((1, RB, D2), lambda r: (0, r, 0)),
            pl.BlockSpec((RB, F), lambda r: (r, 0)),
            pl.BlockSpec((D2 + F, Hout), lambda r: (0, 0)),
            pl.BlockSpec((1, Hout), lambda r: (0, 0)),
        ],
        out_specs=pl.BlockSpec((RB, Hout), lambda r: (r, 0)),
        out_shape=jax.ShapeDtypeStruct((N, Hout), jnp.float32),
    )(h, x, At, b)


def _make_spmm():
    mesh = plsc.VectorSubcoreMesh(core_axis_name="c", subcore_axis_name="s")
    JPE = D2 // 16

    def body(table, cols_h, rows_h, vals_h, zeros_h, flag_h, out,
             cols_v, rows_v, vals_v, gbuf, ridx, fvec, acc, gsem):
        c = lax.axis_index("c")
        s = lax.axis_index("s")
        pltpu.sync_copy(flag_h, fvec)
        f = fvec[...]
        active = jnp.logical_or(c == 0, f[0] != 0)

        @pl.when(active)
        def _():
            pltpu.sync_copy(cols_h.at[c, s], cols_v)
            pltpu.sync_copy(rows_h.at[s], rows_v)
            pltpu.sync_copy(vals_h.at[s], vals_v)
            base = s * RPT

            for p in range(2):
                for z in range(RPT // ZR):
                    pltpu.sync_copy(zeros_h, acc.at[pl.ds(base + z * ZR, ZR)])
                plsc.subcore_barrier()

                def chunk(g, carry, p=p):
                    pltpu.async_copy(table.at[cols_v.at[g]], gbuf, gsem).wait()
                    for m in range(C // 16):
                        sl16 = pl.ds(m * 16, 16)
                        v16 = vals_v[g, sl16]
                        r16 = rows_v[g, sl16]
                        loc = r16 - p * HALF
                        ok = jnp.logical_and(loc >= 0, loc < HALF)
                        mv16 = jnp.where(ok, v16, 0.0)
                        ridx[0, sl16] = jnp.where(ok, loc, 0)
                        for t in range(16):
                            k = m * 16 + t
                            sp = jnp.full((16,), mv16[t], jnp.float32)
                            for j in range(JPE):
                                sl = pl.ds(j * 16, 16)
                                gbuf[k, sl] = gbuf[k, sl] * sp
                    pltpu.sync_copy(gbuf, acc.at[ridx.at[0]], add=True)
                    return carry

                lax.fori_loop(0, NCHUNK, chunk, 0)
                plsc.subcore_barrier()
                for z in range(RPT // ZR):
                    pltpu.sync_copy(
                        acc.at[pl.ds(base + z * ZR, ZR)],
                        out.at[c, pl.ds(p * HALF + base + z * ZR, ZR)])

    return pl.kernel(
        body,
        out_type=jax.ShapeDtypeStruct((2, NP, D2), jnp.float32),
        mesh=mesh,
        scratch_types=[
            pltpu.VMEM((NCHUNK, C), jnp.int32),
            pltpu.VMEM((NCHUNK, C), jnp.int32),
            pltpu.VMEM((NCHUNK, C), jnp.float32),
            pltpu.VMEM((C, D2), jnp.float32),
            pltpu.VMEM((8, C), jnp.int32),
            pltpu.VMEM((16,), jnp.int32),
            pltpu.VMEM_SHARED((HALF, D2), jnp.float32),
            pltpu.SemaphoreType.DMA,
        ],
    )


_spmm = functools.cache(_make_spmm)


def kernel(features, edge_index, edge_values, W_gc0, W_gc1, W_gc2,
           Wt0, bt0, Wt1, bt1, Wt2, bt2):
    x = features
    rows = edge_index[0]
    cols = edge_index[1]
    offs = jnp.arange(2, dtype=jnp.int32) * N
    cols2 = (cols[None, :] + offs[:, None]).reshape(2, TILES, NCHUNK, C)
    rows3 = rows.reshape(TILES, NCHUNK, C)
    vals3 = edge_values.reshape(TILES, NCHUNK, C)
    z128 = jnp.zeros((ZR, D2), jnp.float32)
    f_on = jnp.ones((16,), jnp.int32)
    f_off = jnp.zeros((16,), jnp.int32)

    At0 = Wt0.T
    At1 = Wt1.T
    At2 = Wt2.T
    b0 = bt0.reshape(1, -1)
    b1 = bt1.reshape(1, -1)
    b2 = bt2.reshape(1, -1)

    wg1_3 = W_gc1.reshape(256, 2, D2).transpose(1, 0, 2)
    wg2_3 = jnp.stack([W_gc2, jnp.zeros_like(W_gc2)], axis=0)

    xw0 = _mm_halves(x, W_gc0)
    h1 = _spmm()(xw0, cols2, rows3, vals3, z128, f_on)
    xw1 = _trans_mm_halves(h1, x, At0, b0, wg1_3)
    h2 = _spmm()(xw1, cols2, rows3, vals3, z128, f_on)
    xw2 = _trans_mm_halves(h2, x, At1, b1, wg2_3)
    h3 = _spmm()(xw2, cols2, rows3, vals3, z128, f_off)
    return _trans_final(h3, x, At2, b2)
